# baseline (speedup 1.0000x reference)
import dataclasses
import functools

import jax
import jax.numpy as jnp
from jax import lax
from jax.experimental import pallas as pl
from jax.experimental.pallas import tpu as pltpu
from jax.experimental.pallas import tpu_sc as plsc

_BLK = 2048
_NC = 2
_NS = 16
_LANES = 16


def _heads_block_kernel(h_ref, w_ref, b_ref, tab_ref):
    h_bf = h_ref[...].astype(jnp.bfloat16)
    y = jnp.dot(h_bf, w_ref[...], preferred_element_type=jnp.float32)
    y = y + b_ref[...]
    tab_ref[...] = y.reshape(2 * y.shape[0], 128)


def _all_env_table(h, w_flat, b_flat, n_env, a_dim):
    n, d = h.shape
    blk = _BLK
    grid = n // blk
    return pl.pallas_call(
        _heads_block_kernel,
        grid=(grid,),
        in_specs=[
            pl.BlockSpec((blk, d), lambda i: (i, 0)),
            pl.BlockSpec((d, n_env * a_dim), lambda i: (0, 0)),
            pl.BlockSpec((1, n_env * a_dim), lambda i: (0, 0)),
        ],
        out_specs=pl.BlockSpec((2 * blk, 128), lambda i: (i, 0)),
        out_shape=jax.ShapeDtypeStruct((2 * n, 128), jnp.float32),
        compiler_params=pltpu.CompilerParams(
            dimension_semantics=("arbitrary",),
        ),
    )(h, w_flat, b_flat)


def _sc_dispatch(table, row_idx, col_idx, n, a_dim):
    nw = _NC * _NS
    b_per_w = n // nw
    mesh = plsc.VectorSubcoreMesh(core_axis_name="c", subcore_axis_name="s")
    cp = pltpu.CompilerParams()
    if "needs_layout_passes" in pltpu.CompilerParams.__dataclass_fields__:
        cp = dataclasses.replace(cp, needs_layout_passes=False)

    @functools.partial(
        pl.kernel,
        mesh=mesh,
        compiler_params=cp,
        out_type=jax.ShapeDtypeStruct((n, a_dim), jnp.float32),
        scratch_types=[
            pltpu.VMEM((b_per_w,), jnp.int32),
            pltpu.VMEM((b_per_w, a_dim), jnp.int32),
            pltpu.VMEM((b_per_w, 128), jnp.float32),
            pltpu.VMEM((b_per_w, a_dim), jnp.float32),
            pltpu.SemaphoreType.DMA,
        ],
    )
    def dispatch_kernel(tab_hbm, ridx_hbm, cidx_hbm, out_hbm,
                        ridx_v, cidx_v, rows_v, out_v, sem):
        wid = lax.axis_index("s") * _NC + lax.axis_index("c")
        base = wid * b_per_w
        pltpu.sync_copy(ridx_hbm.at[pl.ds(base, b_per_w)], ridx_v)
        pltpu.sync_copy(cidx_hbm.at[pl.ds(base, b_per_w)], cidx_v)
        pltpu.async_copy(tab_hbm.at[ridx_v], rows_v, sem).wait()

        @pl.loop(0, b_per_w)
        def _(t):
            row = rows_v.at[t]
            for j0 in range(0, a_dim, _LANES):
                cols = cidx_v[t, pl.ds(j0, _LANES)]
                out_v[t, pl.ds(j0, _LANES)] = plsc.load_gather(row, [cols])

        pltpu.sync_copy(out_v, out_hbm.at[pl.ds(base, b_per_w)])

    return dispatch_kernel(table, row_idx, col_idx)


def kernel(h, env_ids, W, b):
    n, d = h.shape
    n_env, _, a_dim = W.shape

    w_flat = W.transpose(1, 0, 2).reshape(d, n_env * a_dim).astype(jnp.bfloat16)
    b_flat = b.reshape(1, n_env * a_dim)
    env = env_ids.reshape(-1).astype(jnp.int32)
    row_idx = jnp.arange(n, dtype=jnp.int32) * 2 + env // 4
    col_idx = (env % 4)[:, None] * a_dim + jnp.arange(a_dim, dtype=jnp.int32)

    table = _all_env_table(h, w_flat, b_flat, n_env, a_dim)
    return _sc_dispatch(table, row_idx, col_idx, n, a_dim)
